# baseline (speedup 1.0000x reference)
import functools

import jax
import jax.numpy as jnp
from jax import lax
from jax.experimental import pallas as pl
from jax.experimental.pallas import tpu as pltpu
from jax.experimental.pallas import tpu_sc as plsc

T = 200
B = 4096
D = 64
VOCAB = 1000000
N = T * B
NC = 2
NS = 16
NW = NC * NS
NPW = N // NW
CHUNK = 256
NCHUNK = NPW // CHUNK
NSTEP = NCHUNK // 2
NBG = CHUNK // 128
NK = CHUNK // 16

_mesh = plsc.VectorSubcoreMesh(core_axis_name="c", subcore_axis_name="s")


@functools.partial(
    pl.kernel,
    out_type=jax.ShapeDtypeStruct((T, 8, B // 128, 8, 128), jnp.float32),
    mesh=_mesh,
    scratch_types=[
        pltpu.VMEM((CHUNK,), jnp.int32),
        pltpu.VMEM((CHUNK,), jnp.int32),
        pltpu.VMEM((CHUNK, D), jnp.float32),
        pltpu.VMEM((CHUNK, D), jnp.float32),
        pltpu.VMEM((8, NBG, 8, 128), jnp.float32),
        pltpu.VMEM((8, NBG, 8, 128), jnp.float32),
        pltpu.SemaphoreType.DMA,
        pltpu.SemaphoreType.DMA,
        pltpu.SemaphoreType.DMA,
        pltpu.SemaphoreType.DMA,
        pltpu.SemaphoreType.DMA,
        pltpu.SemaphoreType.DMA,
    ],
    compiler_params=pltpu.CompilerParams(
        use_tc_tiling_on_sc=False, needs_layout_passes=False),
)
def _gather_kernel(idx_hbm, table_hbm, out_hbm, idx0, idx1, rows0, rows1,
                   tr0, tr1, isem0, isem1, gsem0, gsem1, wsem0, wsem1):
    wid = lax.axis_index("s") * NC + lax.axis_index("c")
    base = wid * NPW
    iota = lax.iota(jnp.int32, 16)
    rregs = [iota + 16 * k for k in range(NK)]

    def start_idx(buf, sem, chunk):
        off = base + lax.min(chunk, NCHUNK - 1) * CHUNK
        pltpu.async_copy(idx_hbm.at[pl.ds(off, CHUNK)], buf, sem)

    def wait_idx(buf, sem):
        pltpu.make_async_copy(idx_hbm.at[pl.ds(base, CHUNK)], buf, sem).wait()

    def start_gather(ibuf, rbuf, sem):
        return pltpu.async_copy(table_hbm.at[ibuf], rbuf, sem)

    def transpose(rbuf, tbuf):
        def dbody(d, carry):
            dg = d // 8
            dr = d % 8
            cvec = jnp.full((16,), 0, jnp.int32) + d
            for kk in range(NK):
                vals = plsc.load_gather(rbuf, [rregs[kk], cvec])
                tbuf[dg, kk // 8, dr, pl.ds((kk % 8) * 16, 16)] = vals
            return carry
        lax.fori_loop(0, D, dbody, 0)

    def start_wb(tbuf, sem, chunk):
        off = base + chunk * CHUNK
        t = off // B
        bg = (off % B) // 128
        pltpu.async_copy(tbuf, out_hbm.at[t, :, pl.ds(bg, NBG), :, :], sem)

    def wait_wb(tbuf, sem):
        pltpu.make_async_copy(
            tbuf, out_hbm.at[0, :, pl.ds(0, NBG), :, :], sem).wait()

    start_idx(idx0, isem0, 0)
    start_idx(idx1, isem1, 1)
    wait_idx(idx0, isem0)
    g0 = start_gather(idx0, rows0, gsem0)
    wait_idx(idx1, isem1)
    g1 = start_gather(idx1, rows1, gsem1)
    g0.wait()
    transpose(rows0, tr0)
    start_wb(tr0, wsem0, 0)
    start_idx(idx0, isem0, 2)
    g1.wait()
    transpose(rows1, tr1)
    start_wb(tr1, wsem1, 1)
    start_idx(idx1, isem1, 3)

    def body(s, carry):
        c0 = 2 * s
        wait_idx(idx0, isem0)
        d0 = start_gather(idx0, rows0, gsem0)
        wait_idx(idx1, isem1)
        d1 = start_gather(idx1, rows1, gsem1)
        d0.wait()
        wait_wb(tr0, wsem0)
        transpose(rows0, tr0)
        start_wb(tr0, wsem0, c0)
        start_idx(idx0, isem0, c0 + 2)
        d1.wait()
        wait_wb(tr1, wsem1)
        transpose(rows1, tr1)
        start_wb(tr1, wsem1, c0 + 1)
        start_idx(idx1, isem1, c0 + 3)
        return carry

    lax.fori_loop(1, NSTEP, body, 0)

    wait_wb(tr0, wsem0)
    wait_wb(tr1, wsem1)
    wait_idx(idx0, isem0)
    wait_idx(idx1, isem1)


def kernel(x, table):
    flat = x.reshape(-1)
    out5 = _gather_kernel(flat, table)
    return (out5.transpose(0, 1, 3, 2, 4)
            .reshape(T, D, B)
            .transpose(0, 2, 1))


# speedup vs baseline: 2.1102x; 2.1102x over previous
import functools

import jax
import jax.numpy as jnp
from jax import lax
from jax.experimental import pallas as pl
from jax.experimental.pallas import tpu as pltpu
from jax.experimental.pallas import tpu_sc as plsc

T = 200
B = 4096
D = 64
VOCAB = 1000000
N = T * B
NC = 2
NS = 16
NW = NC * NS
NPW = N // NW
CHUNK = 512
NCHUNK = NPW // CHUNK
NSTEP = NCHUNK // 2

_mesh = plsc.VectorSubcoreMesh(core_axis_name="c", subcore_axis_name="s")


@functools.partial(
    pl.kernel,
    out_type=jax.ShapeDtypeStruct((T, B, 2 * D), jnp.float32),
    mesh=_mesh,
    scratch_types=[
        pltpu.VMEM((CHUNK,), jnp.int32),
        pltpu.VMEM((CHUNK,), jnp.int32),
        pltpu.VMEM((CHUNK, D), jnp.float32),
        pltpu.VMEM((CHUNK, D), jnp.float32),
        pltpu.SemaphoreType.DMA,
        pltpu.SemaphoreType.DMA,
        pltpu.SemaphoreType.DMA,
        pltpu.SemaphoreType.DMA,
        pltpu.SemaphoreType.DMA,
        pltpu.SemaphoreType.DMA,
    ],
    compiler_params=pltpu.CompilerParams(
        use_tc_tiling_on_sc=False, needs_layout_passes=False),
)
def _gather_kernel(idx_hbm, table_hbm, out_hbm, idx0, idx1, rows0, rows1,
                   isem0, isem1, gsem0, gsem1, wsem0, wsem1):
    wid = lax.axis_index("s") * NC + lax.axis_index("c")
    base = wid * NPW

    def start_idx(buf, sem, chunk):
        off = base + lax.min(chunk, NCHUNK - 1) * CHUNK
        pltpu.async_copy(idx_hbm.at[pl.ds(off, CHUNK)], buf, sem)

    def wait_idx(buf, sem):
        pltpu.make_async_copy(idx_hbm.at[pl.ds(base, CHUNK)], buf, sem).wait()

    def start_gather(ibuf, rbuf, sem):
        return pltpu.async_copy(table_hbm.at[ibuf], rbuf, sem)

    def start_wb(rbuf, sem, chunk):
        off = base + chunk * CHUNK
        t = off // B
        b = off % B
        pltpu.async_copy(rbuf, out_hbm.at[t, pl.ds(b, CHUNK), pl.ds(0, D)],
                         sem)

    def wait_wb(rbuf, sem):
        pltpu.make_async_copy(
            rbuf, out_hbm.at[0, pl.ds(0, CHUNK), pl.ds(0, D)], sem).wait()

    start_idx(idx0, isem0, 0)
    start_idx(idx1, isem1, 1)
    wait_idx(idx0, isem0)
    g0 = start_gather(idx0, rows0, gsem0)
    wait_idx(idx1, isem1)
    g1 = start_gather(idx1, rows1, gsem1)
    g0.wait()
    start_wb(rows0, wsem0, 0)
    start_idx(idx0, isem0, 2)
    g1.wait()
    start_wb(rows1, wsem1, 1)
    start_idx(idx1, isem1, 3)

    def body(s, carry):
        c0 = 2 * s
        wait_idx(idx0, isem0)
        wait_wb(rows0, wsem0)
        d0 = start_gather(idx0, rows0, gsem0)
        wait_idx(idx1, isem1)
        wait_wb(rows1, wsem1)
        d1 = start_gather(idx1, rows1, gsem1)
        d0.wait()
        start_wb(rows0, wsem0, c0)
        start_idx(idx0, isem0, c0 + 2)
        d1.wait()
        start_wb(rows1, wsem1, c0 + 1)
        start_idx(idx1, isem1, c0 + 3)
        return carry

    lax.fori_loop(1, NSTEP, body, 0)

    wait_wb(rows0, wsem0)
    wait_wb(rows1, wsem1)
    wait_idx(idx0, isem0)
    wait_idx(idx1, isem1)


def kernel(x, table):
    flat = x.reshape(-1)
    out128 = _gather_kernel(flat, table)
    return out128[:, :, :D]
